# baseline (speedup 1.0000x reference)
import jax
import jax.numpy as jnp
from jax import lax
from jax.experimental import pallas as pl
from jax.experimental.pallas import tpu as pltpu
from jax.experimental.pallas import tpu_sc as plsc

_NC = 2
_NS = 16
_NW = _NC * _NS
_LANES = 16
_CHUNK = 32


def _add_body(x_ref, p_ref, o_ref):
    o_ref[...] = x_ref[...] + p_ref[...][None, :, :]


def _tc_add(x, pos_emb):
    batch, seq_len, d_model = x.shape
    s_blk = 1024
    grid = (seq_len // s_blk,)
    return pl.pallas_call(
        _add_body,
        grid=grid,
        in_specs=[
            pl.BlockSpec((batch, s_blk, d_model), lambda i: (0, i, 0)),
            pl.BlockSpec((s_blk, d_model), lambda i: (i, 0)),
        ],
        out_specs=pl.BlockSpec((batch, s_blk, d_model), lambda i: (0, i, 0)),
        out_shape=jax.ShapeDtypeStruct((batch, seq_len, d_model), x.dtype),
    )(x, pos_emb)


def _sc_add(x2d, pos_emb, seq_len, d_model):
    n_rows = x2d.shape[0]
    rows_per_w = n_rows // _NW
    n_chunks = rows_per_w // _CHUNK
    mesh = plsc.VectorSubcoreMesh(core_axis_name="c", subcore_axis_name="s")
    n_col_vecs = d_model // _LANES

    @pl.kernel(
        out_type=jax.ShapeDtypeStruct((n_rows, d_model), jnp.float32),
        mesh=mesh,
        scratch_types=[
            pltpu.VMEM((2, _CHUNK, d_model), jnp.float32),
            pltpu.VMEM((2, _CHUNK, d_model), jnp.float32),
            pltpu.SemaphoreType.DMA((2,)),
            pltpu.SemaphoreType.DMA((2,)),
            pltpu.SemaphoreType.DMA((2,)),
        ],
    )
    def run(x_hbm, p_hbm, o_hbm, xbuf, pbuf, sx, sp, so):
        wid = lax.axis_index("s") * _NC + lax.axis_index("c")
        row_base = wid * rows_per_w
        seq_base = lax.rem(wid * rows_per_w, seq_len)

        def start_in(g, slot):
            off = g * _CHUNK
            pltpu.async_copy(
                x_hbm.at[pl.ds(row_base + off, _CHUNK)], xbuf.at[slot], sx.at[slot])
            pltpu.async_copy(
                p_hbm.at[pl.ds(seq_base + off, _CHUNK)], pbuf.at[slot], sp.at[slot])

        start_in(0, 0)

        @pl.loop(0, n_chunks)
        def _g(g):
            slot = lax.rem(g, 2)
            nxt = lax.rem(g + 1, 2)

            @pl.when(g + 1 < n_chunks)
            def _prefetch():
                @pl.when(g >= 1)
                def _drain():
                    pltpu.make_async_copy(
                        xbuf.at[nxt], o_hbm.at[pl.ds(0, _CHUNK)], so.at[nxt]
                    ).wait()

                start_in(g + 1, nxt)

            pltpu.make_async_copy(
                x_hbm.at[pl.ds(0, _CHUNK)], xbuf.at[slot], sx.at[slot]).wait()
            pltpu.make_async_copy(
                p_hbm.at[pl.ds(0, _CHUNK)], pbuf.at[slot], sp.at[slot]).wait()

            @pl.loop(0, _CHUNK)
            def _row(r):
                for ci in range(n_col_vecs):
                    sl = pl.ds(ci * _LANES, _LANES)
                    xbuf[slot, r, sl] = xbuf[slot, r, sl] + pbuf[slot, r, sl]

            off = g * _CHUNK
            pltpu.async_copy(
                xbuf.at[slot], o_hbm.at[pl.ds(row_base + off, _CHUNK)], so.at[slot])

        last = (n_chunks - 1) % 2
        pltpu.make_async_copy(
            xbuf.at[last], o_hbm.at[pl.ds(0, _CHUNK)], so.at[last]
        ).wait()

    return run(x2d, pos_emb)


def kernel(x, pos_emb):
    batch, seq_len, d_model = x.shape
    pe = pos_emb[:seq_len]
    tc_part = _tc_add(x[: batch - 1], pe)
    sc_part = _sc_add(x[batch - 1], pe, seq_len, d_model)
    return jnp.concatenate([tc_part, sc_part[None]], axis=0)


# speedup vs baseline: 3.1897x; 3.1897x over previous
import jax
import jax.numpy as jnp
from jax.experimental import pallas as pl
from jax.experimental.pallas import tpu as pltpu

_R = 1024
_NBUF = 3


def _ring_body(x_hbm, p_hbm, o_hbm, xb, ob, pv, sx, sp, so):
    n_rows = x_hbm.shape[0]
    seq_len = p_hbm.shape[0]
    n_blocks = n_rows // _R
    n_pos_chunks = seq_len // _R

    for k in range(n_pos_chunks):
        pltpu.async_copy(
            p_hbm.at[pl.ds(k * _R, _R)], pv.at[pl.ds(k * _R, _R)], sp.at[k])
    for h in range(_NBUF):
        pltpu.async_copy(x_hbm.at[pl.ds(h * _R, _R)], xb.at[h], sx.at[h])

    for g in range(n_blocks):
        s = g % _NBUF
        pltpu.make_async_copy(
            x_hbm.at[pl.ds(g * _R, _R)], xb.at[s], sx.at[s]).wait()
        if g < n_pos_chunks:
            pltpu.make_async_copy(
                p_hbm.at[pl.ds(g * _R, _R)], pv.at[pl.ds(g * _R, _R)],
                sp.at[g]).wait()
        if g >= _NBUF:
            pltpu.make_async_copy(
                ob.at[s], o_hbm.at[pl.ds((g - _NBUF) * _R, _R)], so.at[s]).wait()
        ob[s] = xb[s] + pv[pl.ds((g % n_pos_chunks) * _R, _R), :]
        pltpu.async_copy(ob.at[s], o_hbm.at[pl.ds(g * _R, _R)], so.at[s])
        h = g + _NBUF
        if h < n_blocks:
            pltpu.async_copy(x_hbm.at[pl.ds(h * _R, _R)], xb.at[s], sx.at[s])

    for t in range(n_blocks - _NBUF, n_blocks):
        s = t % _NBUF
        pltpu.make_async_copy(
            ob.at[s], o_hbm.at[pl.ds(t * _R, _R)], so.at[s]).wait()


def kernel(x, pos_emb):
    batch, seq_len, d_model = x.shape
    x2d = x.reshape(batch * seq_len, d_model)
    pe = pos_emb[:seq_len]
    out = pl.pallas_call(
        _ring_body,
        in_specs=[
            pl.BlockSpec(memory_space=pltpu.HBM),
            pl.BlockSpec(memory_space=pltpu.HBM),
        ],
        out_specs=pl.BlockSpec(memory_space=pltpu.HBM),
        out_shape=jax.ShapeDtypeStruct((batch * seq_len, d_model), x.dtype),
        scratch_shapes=[
            pltpu.VMEM((_NBUF, _R, d_model), jnp.float32),
            pltpu.VMEM((_NBUF, _R, d_model), jnp.float32),
            pltpu.VMEM((seq_len, d_model), jnp.float32),
            pltpu.SemaphoreType.DMA((_NBUF,)),
            pltpu.SemaphoreType.DMA((seq_len // _R,)),
            pltpu.SemaphoreType.DMA((_NBUF,)),
        ],
    )(x2d, pe)
    return out.reshape(batch, seq_len, d_model)
